# baseline (speedup 1.0000x reference)
import jax
import jax.numpy as jnp
from jax import lax
from jax.experimental import pallas as pl
from jax.experimental.pallas import tpu as pltpu

_WS = 7
_NN = 10
_B, _R, _T, _F = 16, 3, 448, 56
_H = _T // _WS
_ROWS = _B * _R * _H
_D = _F
_NF = 7
_SEQ = _H
_BATCH = _B
_GDIM = 168
_HID = 56
_SCORE_NORM = _NN * _D * _H * _R

_SORT7 = [(1, 2), (3, 4), (5, 6), (0, 2), (3, 5), (4, 6), (0, 1), (4, 5),
          (2, 6), (0, 4), (1, 5), (0, 3), (2, 5), (1, 3), (2, 4), (2, 3)]


def _selection_prog():
    n = 64
    comps = []
    k = 2
    while k <= n:
        j = k // 2
        while j >= 1:
            for i in range(n):
                l = i ^ j
                if l > i:
                    comps.append((i, l, (i & k) == 0))
            j //= 2
        k *= 2
    inf = [False] * _D + [True] * (n - _D)
    prog = []
    for (i, l, up) in comps:
        ai, bi = inf[i], inf[l]
        if ai and bi:
            continue
        if up:
            if bi:
                continue
            if ai:
                prog.append((i, l, "swap"))
                inf[i], inf[l] = False, True
                continue
            prog.append((i, l, "up"))
        else:
            if ai:
                continue
            if bi:
                prog.append((i, l, "swap"))
                inf[i], inf[l] = True, False
                continue
            prog.append((i, l, "dn"))
    needed = {_NN - 1, 2 * _NN - 1}
    kept = []
    for op in reversed(prog):
        i, l, _ = op
        if i in needed or l in needed:
            kept.append(op)
            needed.add(i)
            needed.add(l)
    kept.reverse()
    return kept


_SELPROG = _selection_prog()


def _sort7(vals):
    v = list(vals)
    for i, j in _SORT7:
        lo = jnp.minimum(v[i], v[j])
        hi = jnp.maximum(v[i], v[j])
        v[i], v[j] = lo, hi
    return v


def _fused_body(x_ref, sc_ref, o_ref):
    w = [jnp.swapaxes(x_ref[:, s, :], 0, 1) for s in range(_WS)]
    amax = w[0]
    amin = w[0]
    ssum = w[0]
    for s in range(1, _WS):
        amax = jnp.maximum(amax, w[s])
        amin = jnp.minimum(amin, w[s])
        ssum = ssum + w[s]
    mu = ssum * (1.0 / _WS)
    dev = [ws - mu for ws in w]
    ss2 = dev[0] * dev[0]
    for s in range(1, _WS):
        ss2 = ss2 + dev[s] * dev[s]
    astd = jnp.sqrt(ss2 * (1.0 / (_WS - 1)))
    c2 = ss2 * (1.0 / _WS)
    s3 = dev[0] * dev[0] * dev[0]
    s4 = dev[0] * dev[0] * dev[0] * dev[0]
    for s in range(1, _WS):
        d2 = dev[s] * dev[s]
        s3 = s3 + d2 * dev[s]
        s4 = s4 + d2 * d2
    c3 = s3 * (1.0 / _WS)
    c4 = s4 * (1.0 / _WS)
    skew = c3 / (c2 * jnp.sqrt(c2))
    kurt = c4 / (c2 * c2) - 3.0
    med = _sort7(w)[3]
    mad = _sort7([jnp.abs(ws - med) for ws in w])[3]
    A = [amax, amin, mu, astd, skew, kurt, mad]
    for f in range(_NF):
        o_ref[f] = A[f]

    dist = None
    for f in range(_NF):
        df = A[f][:, None, :] - A[f][None, :, :]
        sq = df * df
        dist = sq if dist is None else dist + sq
    s = [dist[p] for p in range(_D)] + [None] * 8
    for (i, l, kind) in _SELPROG:
        if kind == "swap":
            s[i], s[l] = s[l], s[i]
        elif kind == "up":
            a, b = s[i], s[l]
            s[i], s[l] = jnp.minimum(a, b), jnp.maximum(a, b)
        else:
            a, b = s[i], s[l]
            s[i], s[l] = jnp.maximum(a, b), jnp.minimum(a, b)
    v_hit = s[_NN - 1][None, :, :]
    v_miss = s[2 * _NN - 1][None, :, :]
    w = jnp.where(dist <= v_miss, 1.0, 0.0) - 2.0 * jnp.where(
        dist <= v_hit, 1.0, 0.0)

    @pl.when(pl.program_id(0) == 0)
    def _():
        sc_ref[...] = jnp.zeros_like(sc_ref)

    for f in range(_NF):
        absdf = jnp.abs(A[f][:, None, :] - A[f][None, :, :])
        sf = jnp.sum(w * absdf)
        sc_ref[f] = sc_ref[f] + sf


def _gru_body(st_ref, wih_ref, whh_ref, wfc_ref, bih_ref, bhh_ref,
              bfc_ref, wc_ref, bc_ref, sc_ref, o_ref, gi_ref, hall_ref):
    dn = (((1,), (1,)), ((), ()))
    stm = st_ref[...].reshape(_NF * _F, _ROWS)
    gi = None
    for k in range(3):
        vk = None
        for c in range(3):
            term = wc_ref[c, k] * wih_ref[c]
            vk = term if vk is None else vk + term
        p = lax.dot_general(stm, vk, (((0,), (0,)), ((), ())),
                            preferred_element_type=jnp.float32)
        pk = p.reshape(_BATCH * _SEQ, 3, _GDIM)[:, k, :]
        gi = pk if gi is None else gi + pk
    bias = bih_ref[...]
    ones = jnp.ones((1, _NF * _F), jnp.float32)
    for c in range(3):
        rs = lax.dot_general(ones, wih_ref[c], (((1,), (0,)), ((), ())),
                             preferred_element_type=jnp.float32)
        bias = bias + bc_ref[c] * rs
    gi_ref[...] = (gi + bias).reshape(_BATCH, _SEQ, _GDIM)

    whh = whh_ref[...]
    bhh = bhh_ref[...]

    h = jnp.zeros((_BATCH, _HID), jnp.float32)
    for t in range(_SEQ):
        git = gi_ref[:, t, :]
        gh = lax.dot_general(h, whh, dn,
                             preferred_element_type=jnp.float32) + bhh
        r = jax.nn.sigmoid(git[:, 0:56] + gh[:, 0:56])
        z = jax.nn.sigmoid(git[:, 56:112] + gh[:, 56:112])
        n = jnp.tanh(git[:, 112:168] + r * gh[:, 112:168])
        h = (1.0 - z) * n + z * h
        hall_ref[:, t, :] = h

    inv = 1.0 / float(_SCORE_NORM)
    weff = None
    beff = None
    for f in range(_NF):
        sf = sc_ref[f] * inv
        tw = sf * wfc_ref[:, f, :]
        tb = sf * bfc_ref[f:f + 1, :]
        weff = tw if weff is None else weff + tw
        beff = tb if beff is None else beff + tb
    hv = hall_ref[...].reshape(_BATCH * _SEQ, _HID)
    o_ref[...] = lax.dot_general(hv, weff, dn,
                                 preferred_element_type=jnp.float32) + beff


def kernel(x, y, Wc, bc, Wih, Whh, bih, bhh, Wfc, bfc):
    del y
    x3 = x.reshape(_ROWS, _WS, _F)
    tb = 128
    raw, st = pl.pallas_call(
        _fused_body,
        grid=(_ROWS // tb,),
        in_specs=[pl.BlockSpec((tb, _WS, _F), lambda i: (i, 0, 0))],
        out_specs=[pl.BlockSpec((_NF, 8, 128), lambda i: (0, 0, 0)),
                   pl.BlockSpec((_NF, _F, tb), lambda i: (0, 0, i))],
        out_shape=[jax.ShapeDtypeStruct((_NF, 8, 128), jnp.float32),
                   jax.ShapeDtypeStruct((_NF, _F, _ROWS), jnp.float32)],
        compiler_params=pltpu.CompilerParams(
            dimension_semantics=("arbitrary",)),
    )(x3)
    scores = raw[:, 0, 0]

    wihp = Wih.reshape(_GDIM, 3, _F, _NF).transpose(1, 3, 2, 0).reshape(
        3, _NF * _F, _GDIM)

    smem = pl.BlockSpec(memory_space=pltpu.SMEM)
    vmem = pl.BlockSpec(memory_space=pltpu.VMEM)
    out_bm = pl.pallas_call(
        _gru_body,
        in_specs=[vmem, vmem, vmem, vmem, vmem, vmem, vmem,
                  smem, smem, smem],
        out_specs=vmem,
        out_shape=jax.ShapeDtypeStruct((_BATCH * _SEQ, _GDIM), jnp.float32),
        scratch_shapes=[
            pltpu.VMEM((_BATCH, _SEQ, _GDIM), jnp.float32),
            pltpu.VMEM((_BATCH, _SEQ, _HID), jnp.float32),
        ],
    )(st, wihp, Whh, Wfc.reshape(_GDIM, _NF, _HID),
      bih.reshape(1, _GDIM), bhh.reshape(1, _GDIM),
      bfc.reshape(_GDIM, _NF).T, Wc, bc, scores)

    return out_bm.reshape(_BATCH, _SEQ, _R, _HID)


# speedup vs baseline: 1.0609x; 1.0609x over previous
import jax
import jax.numpy as jnp
from jax import lax
from jax.experimental import pallas as pl
from jax.experimental.pallas import tpu as pltpu

_WS = 7
_NN = 10
_B, _R, _T, _F = 16, 3, 448, 56
_H = _T // _WS
_ROWS = _B * _R * _H
_D = _F
_NF = 7
_SEQ = _H
_BATCH = _B
_GDIM = 168
_HID = 56
_SCORE_NORM = _NN * _D * _H * _R

_SORT7 = [(1, 2), (3, 4), (5, 6), (0, 2), (3, 5), (4, 6), (0, 1), (4, 5),
          (2, 6), (0, 4), (1, 5), (0, 3), (2, 5), (1, 3), (2, 4), (2, 3)]


def _selection_prog():
    n = 64
    comps = []
    k = 2
    while k <= n:
        j = k // 2
        while j >= 1:
            for i in range(n):
                l = i ^ j
                if l > i:
                    comps.append((i, l, (i & k) == 0))
            j //= 2
        k *= 2
    inf = [False] * _D + [True] * (n - _D)
    prog = []
    for (i, l, up) in comps:
        ai, bi = inf[i], inf[l]
        if ai and bi:
            continue
        if up:
            if bi:
                continue
            if ai:
                prog.append((i, l, "swap"))
                inf[i], inf[l] = False, True
                continue
            prog.append((i, l, "up"))
        else:
            if ai:
                continue
            if bi:
                prog.append((i, l, "swap"))
                inf[i], inf[l] = True, False
                continue
            prog.append((i, l, "dn"))
    needed = {_NN - 1, 2 * _NN - 1}
    kept = []
    for op in reversed(prog):
        i, l, _ = op
        if i in needed or l in needed:
            kept.append(op)
            needed.add(i)
            needed.add(l)
    kept.reverse()
    return kept


_SELPROG = _selection_prog()


def _sort7(vals):
    v = list(vals)
    for i, j in _SORT7:
        lo = jnp.minimum(v[i], v[j])
        hi = jnp.maximum(v[i], v[j])
        v[i], v[j] = lo, hi
    return v


def _fused_body(xw_ref, sc_ref, o_ref):
    w = [xw_ref[s] for s in range(_WS)]
    amax = w[0]
    amin = w[0]
    ssum = w[0]
    for s in range(1, _WS):
        amax = jnp.maximum(amax, w[s])
        amin = jnp.minimum(amin, w[s])
        ssum = ssum + w[s]
    mu = ssum * (1.0 / _WS)
    dev = [ws - mu for ws in w]
    ss2 = dev[0] * dev[0]
    for s in range(1, _WS):
        ss2 = ss2 + dev[s] * dev[s]
    astd = jnp.sqrt(ss2 * (1.0 / (_WS - 1)))
    c2 = ss2 * (1.0 / _WS)
    s3 = dev[0] * dev[0] * dev[0]
    s4 = dev[0] * dev[0] * dev[0] * dev[0]
    for s in range(1, _WS):
        d2 = dev[s] * dev[s]
        s3 = s3 + d2 * dev[s]
        s4 = s4 + d2 * d2
    c3 = s3 * (1.0 / _WS)
    c4 = s4 * (1.0 / _WS)
    skew = c3 / (c2 * jnp.sqrt(c2))
    kurt = c4 / (c2 * c2) - 3.0
    med = _sort7(w)[3]
    mad = _sort7([jnp.abs(ws - med) for ws in w])[3]
    A = [amax, amin, mu, astd, skew, kurt, mad]
    for f in range(_NF):
        o_ref[f] = A[f]

    dist = None
    for f in range(_NF):
        df = A[f][:, None, :] - A[f][None, :, :]
        sq = df * df
        dist = sq if dist is None else dist + sq
    s = [dist[p] for p in range(_D)] + [None] * 8
    for (i, l, kind) in _SELPROG:
        if kind == "swap":
            s[i], s[l] = s[l], s[i]
        elif kind == "up":
            a, b = s[i], s[l]
            s[i], s[l] = jnp.minimum(a, b), jnp.maximum(a, b)
        else:
            a, b = s[i], s[l]
            s[i], s[l] = jnp.maximum(a, b), jnp.minimum(a, b)
    v_hit = s[_NN - 1][None, :, :]
    v_miss = s[2 * _NN - 1][None, :, :]
    w = jnp.where(dist <= v_miss, 1.0, 0.0) - 2.0 * jnp.where(
        dist <= v_hit, 1.0, 0.0)

    @pl.when(pl.program_id(0) == 0)
    def _():
        sc_ref[...] = jnp.zeros_like(sc_ref)

    for f in range(_NF):
        absdf = jnp.abs(A[f][:, None, :] - A[f][None, :, :])
        sf = jnp.sum(w * absdf)
        sc_ref[f] = sc_ref[f] + sf


def _gru_body(st_ref, wih_ref, whh_ref, wfc_ref, bih_ref, bhh_ref,
              bfc_ref, wc_ref, bc_ref, sc_ref, o_ref, gi_ref, hall_ref):
    dn = (((1,), (1,)), ((), ()))
    stm = st_ref[...].reshape(_NF * _F, _ROWS)
    gi = None
    for k in range(3):
        vk = None
        for c in range(3):
            term = wc_ref[c, k] * wih_ref[c]
            vk = term if vk is None else vk + term
        p = lax.dot_general(stm, vk, (((0,), (0,)), ((), ())),
                            preferred_element_type=jnp.float32)
        pk = p.reshape(_BATCH * _SEQ, 3, _GDIM)[:, k, :]
        gi = pk if gi is None else gi + pk
    bias = bih_ref[...]
    ones = jnp.ones((1, _NF * _F), jnp.float32)
    for c in range(3):
        rs = lax.dot_general(ones, wih_ref[c], (((1,), (0,)), ((), ())),
                             preferred_element_type=jnp.float32)
        bias = bias + bc_ref[c] * rs
    gi_ref[...] = (gi + bias).reshape(_BATCH, _SEQ, _GDIM)

    whh = whh_ref[...]
    bhh = bhh_ref[...]

    h = jnp.zeros((_BATCH, _HID), jnp.float32)
    for t in range(_SEQ):
        git = gi_ref[:, t, :]
        gh = lax.dot_general(h, whh, dn,
                             preferred_element_type=jnp.float32) + bhh
        r = jax.nn.sigmoid(git[:, 0:56] + gh[:, 0:56])
        z = jax.nn.sigmoid(git[:, 56:112] + gh[:, 56:112])
        n = jnp.tanh(git[:, 112:168] + r * gh[:, 112:168])
        h = (1.0 - z) * n + z * h
        hall_ref[:, t, :] = h

    inv = 1.0 / float(_SCORE_NORM)
    weff = None
    beff = None
    for f in range(_NF):
        sf = sc_ref[f] * inv
        tw = sf * wfc_ref[:, f, :]
        tb = sf * bfc_ref[f:f + 1, :]
        weff = tw if weff is None else weff + tw
        beff = tb if beff is None else beff + tb
    hv = hall_ref[...].reshape(_BATCH * _SEQ, _HID)
    o_ref[...] = lax.dot_general(hv, weff, dn,
                                 preferred_element_type=jnp.float32) + beff


def kernel(x, y, Wc, bc, Wih, Whh, bih, bhh, Wfc, bfc):
    del y
    xw = x.reshape(_ROWS, _WS, _F).transpose(1, 2, 0)
    tb = 256
    raw, st = pl.pallas_call(
        _fused_body,
        grid=(_ROWS // tb,),
        in_specs=[pl.BlockSpec((_WS, _F, tb), lambda i: (0, 0, i))],
        out_specs=[pl.BlockSpec((_NF, 8, 128), lambda i: (0, 0, 0)),
                   pl.BlockSpec((_NF, _F, tb), lambda i: (0, 0, i))],
        out_shape=[jax.ShapeDtypeStruct((_NF, 8, 128), jnp.float32),
                   jax.ShapeDtypeStruct((_NF, _F, _ROWS), jnp.float32)],
        compiler_params=pltpu.CompilerParams(
            dimension_semantics=("arbitrary",)),
    )(xw)
    scores = raw[:, 0, 0]

    wihp = Wih.reshape(_GDIM, 3, _F, _NF).transpose(1, 3, 2, 0).reshape(
        3, _NF * _F, _GDIM)

    smem = pl.BlockSpec(memory_space=pltpu.SMEM)
    vmem = pl.BlockSpec(memory_space=pltpu.VMEM)
    out_bm = pl.pallas_call(
        _gru_body,
        in_specs=[vmem, vmem, vmem, vmem, vmem, vmem, vmem,
                  smem, smem, smem],
        out_specs=vmem,
        out_shape=jax.ShapeDtypeStruct((_BATCH * _SEQ, _GDIM), jnp.float32),
        scratch_shapes=[
            pltpu.VMEM((_BATCH, _SEQ, _GDIM), jnp.float32),
            pltpu.VMEM((_BATCH, _SEQ, _HID), jnp.float32),
        ],
    )(st, wihp, Whh, Wfc.reshape(_GDIM, _NF, _HID),
      bih.reshape(1, _GDIM), bhh.reshape(1, _GDIM),
      bfc.reshape(_GDIM, _NF).T, Wc, bc, scores)

    return out_bm.reshape(_BATCH, _SEQ, _R, _HID)


# speedup vs baseline: 1.0912x; 1.0286x over previous
import jax
import jax.numpy as jnp
from jax import lax
from jax.experimental import pallas as pl
from jax.experimental.pallas import tpu as pltpu

_WS = 7
_NN = 10
_B, _R, _T, _F = 16, 3, 448, 56
_H = _T // _WS
_ROWS = _B * _R * _H
_D = _F
_NF = 7
_SEQ = _H
_BATCH = _B
_GDIM = 168
_HID = 56
_SCORE_NORM = _NN * _D * _H * _R

_SORT7 = [(1, 2), (3, 4), (5, 6), (0, 2), (3, 5), (4, 6), (0, 1), (4, 5),
          (2, 6), (0, 4), (1, 5), (0, 3), (2, 5), (1, 3), (2, 4), (2, 3)]


def _selection_prog():
    n = 64
    comps = []
    k = 2
    while k <= n:
        j = k // 2
        while j >= 1:
            for i in range(n):
                l = i ^ j
                if l > i:
                    comps.append((i, l, (i & k) == 0))
            j //= 2
        k *= 2
    inf = [False] * _D + [True] * (n - _D)
    prog = []
    for (i, l, up) in comps:
        ai, bi = inf[i], inf[l]
        if ai and bi:
            continue
        if up:
            if bi:
                continue
            if ai:
                prog.append((i, l, "swap"))
                inf[i], inf[l] = False, True
                continue
            prog.append((i, l, "up"))
        else:
            if ai:
                continue
            if bi:
                prog.append((i, l, "swap"))
                inf[i], inf[l] = True, False
                continue
            prog.append((i, l, "dn"))
    needed = {_NN - 1, 2 * _NN - 1}
    kept = []
    for op in reversed(prog):
        i, l, _ = op
        if i in needed or l in needed:
            kept.append(op)
            needed.add(i)
            needed.add(l)
    kept.reverse()
    return kept


_SELPROG = _selection_prog()


def _sort7(vals):
    v = list(vals)
    for i, j in _SORT7:
        lo = jnp.minimum(v[i], v[j])
        hi = jnp.maximum(v[i], v[j])
        v[i], v[j] = lo, hi
    return v


def _fused_body(xw_ref, sc_ref, o_ref):
    w = [xw_ref[s] for s in range(_WS)]
    amax = w[0]
    amin = w[0]
    ssum = w[0]
    for s in range(1, _WS):
        amax = jnp.maximum(amax, w[s])
        amin = jnp.minimum(amin, w[s])
        ssum = ssum + w[s]
    mu = ssum * (1.0 / _WS)
    dev = [ws - mu for ws in w]
    ss2 = dev[0] * dev[0]
    for s in range(1, _WS):
        ss2 = ss2 + dev[s] * dev[s]
    astd = jnp.sqrt(ss2 * (1.0 / (_WS - 1)))
    c2 = ss2 * (1.0 / _WS)
    s3 = dev[0] * dev[0] * dev[0]
    s4 = dev[0] * dev[0] * dev[0] * dev[0]
    for s in range(1, _WS):
        d2 = dev[s] * dev[s]
        s3 = s3 + d2 * dev[s]
        s4 = s4 + d2 * d2
    c3 = s3 * (1.0 / _WS)
    c4 = s4 * (1.0 / _WS)
    skew = c3 / (c2 * jnp.sqrt(c2))
    kurt = c4 / (c2 * c2) - 3.0
    med = _sort7(w)[3]
    mad = _sort7([jnp.abs(ws - med) for ws in w])[3]
    A = [amax, amin, mu, astd, skew, kurt, mad]
    for f in range(_NF):
        o_ref[f] = A[f]

    dist = None
    for f in range(_NF):
        df = A[f][:, None, :] - A[f][None, :, :]
        sq = df * df
        dist = sq if dist is None else dist + sq
    s = [dist[p] for p in range(_D)] + [None] * 8
    for (i, l, kind) in _SELPROG:
        if kind == "swap":
            s[i], s[l] = s[l], s[i]
        elif kind == "up":
            a, b = s[i], s[l]
            s[i], s[l] = jnp.minimum(a, b), jnp.maximum(a, b)
        else:
            a, b = s[i], s[l]
            s[i], s[l] = jnp.maximum(a, b), jnp.minimum(a, b)
    v_hit = s[_NN - 1][None, :, :]
    v_miss = s[2 * _NN - 1][None, :, :]
    w = jnp.where(dist <= v_miss, 1.0, 0.0) - 2.0 * jnp.where(
        dist <= v_hit, 1.0, 0.0)

    @pl.when(pl.program_id(0) == 0)
    def _():
        sc_ref[...] = jnp.zeros_like(sc_ref)

    for f in range(_NF):
        absdf = jnp.abs(A[f][:, None, :] - A[f][None, :, :])
        sf = jnp.sum(w * absdf)
        sc_ref[f] = sc_ref[f] + sf


def _gru_body(st_ref, wih_ref, whh_ref, wfc_ref, bih_ref, bhh_ref,
              bfc_ref, wc_ref, bc_ref, sc_ref, o_ref, gi_ref, hall_ref):
    dn = (((1,), (1,)), ((), ()))
    stm = st_ref[...].reshape(_NF * _F, _ROWS)
    gi = None
    for k in range(3):
        vk = None
        for c in range(3):
            term = wc_ref[c, k] * wih_ref[c]
            vk = term if vk is None else vk + term
        p = lax.dot_general(stm, vk, (((0,), (0,)), ((), ())),
                            preferred_element_type=jnp.float32)
        pk = p.reshape(_BATCH * _SEQ, 3, _GDIM)[:, k, :]
        gi = pk if gi is None else gi + pk
    bias = bih_ref[...]
    ones = jnp.ones((1, _NF * _F), jnp.float32)
    for c in range(3):
        rs = lax.dot_general(ones, wih_ref[c], (((1,), (0,)), ((), ())),
                             preferred_element_type=jnp.float32)
        bias = bias + bc_ref[c] * rs
    gi_ref[...] = (gi + bias).reshape(_BATCH, _SEQ, _GDIM)

    whh = whh_ref[...]
    bhh = bhh_ref[...]

    h = jnp.zeros((_BATCH, _HID), jnp.float32)
    for t in range(_SEQ):
        git = gi_ref[:, t, :]
        gh = lax.dot_general(h, whh, dn,
                             preferred_element_type=jnp.float32) + bhh
        r = jax.nn.sigmoid(git[:, 0:56] + gh[:, 0:56])
        z = jax.nn.sigmoid(git[:, 56:112] + gh[:, 56:112])
        n = jnp.tanh(git[:, 112:168] + r * gh[:, 112:168])
        h = (1.0 - z) * n + z * h
        hall_ref[:, t, :] = h

    inv = 1.0 / float(_SCORE_NORM)
    weff = None
    beff = None
    for f in range(_NF):
        sf = sc_ref[f] * inv
        tw = sf * wfc_ref[:, f, :]
        tb = sf * bfc_ref[f:f + 1, :]
        weff = tw if weff is None else weff + tw
        beff = tb if beff is None else beff + tb
    hv = hall_ref[...].reshape(_BATCH * _SEQ, _HID)
    o_ref[...] = lax.dot_general(hv, weff, dn,
                                 preferred_element_type=jnp.float32) + beff


def kernel(x, y, Wc, bc, Wih, Whh, bih, bhh, Wfc, bfc):
    del y
    xw = x.reshape(_ROWS, _WS, _F).transpose(1, 2, 0)
    tb = 128
    raw, st = pl.pallas_call(
        _fused_body,
        grid=(_ROWS // tb,),
        in_specs=[pl.BlockSpec((_WS, _F, tb), lambda i: (0, 0, i))],
        out_specs=[pl.BlockSpec((_NF, 8, 128), lambda i: (0, 0, 0)),
                   pl.BlockSpec((_NF, _F, tb), lambda i: (0, 0, i))],
        out_shape=[jax.ShapeDtypeStruct((_NF, 8, 128), jnp.float32),
                   jax.ShapeDtypeStruct((_NF, _F, _ROWS), jnp.float32)],
        compiler_params=pltpu.CompilerParams(
            dimension_semantics=("arbitrary",)),
    )(xw)
    scores = raw[:, 0, 0]

    wihp = Wih.reshape(_GDIM, 3, _F, _NF).transpose(1, 3, 2, 0).reshape(
        3, _NF * _F, _GDIM)

    smem = pl.BlockSpec(memory_space=pltpu.SMEM)
    vmem = pl.BlockSpec(memory_space=pltpu.VMEM)
    out_bm = pl.pallas_call(
        _gru_body,
        in_specs=[vmem, vmem, vmem, vmem, vmem, vmem, vmem,
                  smem, smem, smem],
        out_specs=vmem,
        out_shape=jax.ShapeDtypeStruct((_BATCH * _SEQ, _GDIM), jnp.float32),
        scratch_shapes=[
            pltpu.VMEM((_BATCH, _SEQ, _GDIM), jnp.float32),
            pltpu.VMEM((_BATCH, _SEQ, _HID), jnp.float32),
        ],
    )(st, wihp, Whh, Wfc.reshape(_GDIM, _NF, _HID),
      bih.reshape(1, _GDIM), bhh.reshape(1, _GDIM),
      bfc.reshape(_GDIM, _NF).T, Wc, bc, scores)

    return out_bm.reshape(_BATCH, _SEQ, _R, _HID)
